# baseline (speedup 1.0000x reference)
import jax
import jax.numpy as jnp
import numpy as np
from jax.experimental import pallas as pl
from jax.experimental.pallas import tpu as pltpu

R = 32
D = 512
K = 6
M_REG = 8
FB_ALPHA = 0.1
FB_SCALE = 1.0 / np.sqrt(M_REG)


def _router_kernel(nbr_smem, h_ref, w_ref, coords_ref, wreg_ref, betas_ref,
                   mask_ref, nbr_col_ref, out_ref, coeff_ref):
    r = pl.program_id(0)
    k = pl.program_id(1)

    @pl.when((r == 0) & (k == 0))
    def _compute_coeff():
        lane = jax.lax.broadcasted_iota(jnp.int32, (R * K, R), 1)
        nbr = nbr_col_ref[...]
        oh_nbr = (lane == nbr).astype(jnp.float32)
        own = jax.lax.broadcasted_iota(jnp.int32, (R * K, R), 0) // K
        oh_own = (lane == own).astype(jnp.float32)
        delta = jax.lax.dot_general(
            oh_own - oh_nbr, coords_ref[...],
            (((1,), (0,)), ((), ())), preferred_element_type=jnp.float32,
        )
        S = jax.lax.dot_general(
            delta, wreg_ref[...],
            (((1,), (1,)), ((), ())), preferred_element_type=jnp.float32,
        )
        fb = (jnp.cos(S) * betas_ref[0:1, :]
              + jnp.sin(S) * betas_ref[1:2, :])
        b = jnp.sum(fb, axis=1, keepdims=True)
        maskN = jax.lax.dot_general(
            oh_nbr, mask_ref[...],
            (((1,), (0,)), ((), ())), preferred_element_type=jnp.float32,
        )
        coeff_ref[...] = (1.0 + (FB_ALPHA * FB_SCALE) * b) * maskN

    idx = nbr_smem[r, k]
    h = h_ref[pl.ds(idx, 1), :]
    y = jax.lax.dot_general(
        h, w_ref[0, 0],
        (((1,), (1,)), ((), ())), preferred_element_type=jnp.float32,
    )
    contrib = y * coeff_ref[pl.ds(r * K + k, 1), :]

    @pl.when(k == 0)
    def _init():
        out_ref[0] = contrib

    @pl.when(k != 0)
    def _accum():
        out_ref[0] += contrib


def kernel(H, reg_mask_prev, reg_coords, W_edge, W_reg, beta_cos, beta_sin, neighbors):
    betas = jnp.stack([beta_cos, beta_sin])
    mask_col = reg_mask_prev.astype(jnp.float32).reshape(R, 1)
    nbr_col = neighbors.reshape(R * K, 1)

    out = pl.pallas_call(
        _router_kernel,
        grid=(R, K),
        in_specs=[
            pl.BlockSpec(memory_space=pltpu.SMEM),
            pl.BlockSpec((R, D), lambda r, k: (0, 0)),
            pl.BlockSpec((1, 1, D, D), lambda r, k: (r, k, 0, 0)),
            pl.BlockSpec((R, 2), lambda r, k: (0, 0)),
            pl.BlockSpec((M_REG, 2), lambda r, k: (0, 0)),
            pl.BlockSpec((2, M_REG), lambda r, k: (0, 0)),
            pl.BlockSpec((R, 1), lambda r, k: (0, 0)),
            pl.BlockSpec((R * K, 1), lambda r, k: (0, 0)),
        ],
        out_specs=pl.BlockSpec((1, 1, D), lambda r, k: (r, 0, 0)),
        out_shape=jax.ShapeDtypeStruct((R, 1, D), jnp.float32),
        scratch_shapes=[pltpu.VMEM((R * K, 1), jnp.float32)],
        compiler_params=pltpu.CompilerParams(
            dimension_semantics=("arbitrary", "arbitrary"),
        ),
    )(neighbors, H, W_edge, reg_coords, W_reg, betas, mask_col, nbr_col)
    return out.reshape(R, D)


# speedup vs baseline: 2.2791x; 2.2791x over previous
import jax
import jax.numpy as jnp
import numpy as np
from jax.experimental import pallas as pl
from jax.experimental.pallas import tpu as pltpu

R = 32
D = 512
K = 6
M_REG = 8
FB_ALPHA = 0.1
FB_SCALE = 1.0 / np.sqrt(M_REG)


RB = 4


def _router_kernel(nbr_smem, h_ref, w_ref, coords_ref, wreg_ref, betas_ref,
                   mask_ref, nbr_col_ref, out_ref, coeff_ref):
    g = pl.program_id(0)

    @pl.when(g == 0)
    def _compute_coeff():
        lane = jax.lax.broadcasted_iota(jnp.int32, (R * K, R), 1)
        nbr = nbr_col_ref[...]
        oh_nbr = (lane == nbr).astype(jnp.float32)
        own = jax.lax.broadcasted_iota(jnp.int32, (R * K, R), 0) // K
        oh_own = (lane == own).astype(jnp.float32)
        delta = jax.lax.dot_general(
            oh_own - oh_nbr, coords_ref[...],
            (((1,), (0,)), ((), ())), preferred_element_type=jnp.float32,
        )
        S = jax.lax.dot_general(
            delta, wreg_ref[...],
            (((1,), (1,)), ((), ())), preferred_element_type=jnp.float32,
        )
        fb = (jnp.cos(S) * betas_ref[0:1, :]
              + jnp.sin(S) * betas_ref[1:2, :])
        b = jnp.sum(fb, axis=1, keepdims=True)
        maskN = jax.lax.dot_general(
            oh_nbr, mask_ref[...],
            (((1,), (0,)), ((), ())), preferred_element_type=jnp.float32,
        )
        coeff_ref[...] = (1.0 + (FB_ALPHA * FB_SCALE) * b) * maskN

    for rb in range(RB):
        r = g * RB + rb
        acc = jnp.zeros((1, D), dtype=jnp.float32)
        for k in range(K):
            idx = nbr_smem[r, k]
            h = h_ref[pl.ds(idx, 1), :]
            y = jax.lax.dot_general(
                h, w_ref[rb, k],
                (((1,), (1,)), ((), ())), preferred_element_type=jnp.float32,
            )
            acc = acc + y * coeff_ref[pl.ds(r * K + k, 1), :]
        out_ref[rb] = acc


def kernel(H, reg_mask_prev, reg_coords, W_edge, W_reg, beta_cos, beta_sin, neighbors):
    betas = jnp.stack([beta_cos, beta_sin])
    mask_col = reg_mask_prev.astype(jnp.float32).reshape(R, 1)
    nbr_col = neighbors.reshape(R * K, 1)

    out = pl.pallas_call(
        _router_kernel,
        grid=(R // RB,),
        in_specs=[
            pl.BlockSpec(memory_space=pltpu.SMEM),
            pl.BlockSpec((R, D), lambda g: (0, 0)),
            pl.BlockSpec((RB, K, D, D), lambda g: (g, 0, 0, 0)),
            pl.BlockSpec((R, 2), lambda g: (0, 0)),
            pl.BlockSpec((M_REG, 2), lambda g: (0, 0)),
            pl.BlockSpec((2, M_REG), lambda g: (0, 0)),
            pl.BlockSpec((R, 1), lambda g: (0, 0)),
            pl.BlockSpec((R * K, 1), lambda g: (0, 0)),
        ],
        out_specs=pl.BlockSpec((RB, 1, D), lambda g: (g, 0, 0)),
        out_shape=jax.ShapeDtypeStruct((R, 1, D), jnp.float32),
        scratch_shapes=[pltpu.VMEM((R * K, 1), jnp.float32)],
        compiler_params=pltpu.CompilerParams(
            dimension_semantics=("arbitrary",),
        ),
    )(neighbors, H, W_edge, reg_coords, W_reg, betas, mask_col, nbr_col)
    return out.reshape(R, D)
